# Initial kernel scaffold; baseline (speedup 1.0000x reference)
import functools

import jax
import jax.numpy as jnp
from jax import lax
from jax.experimental import pallas as pl
from jax.experimental.pallas import tpu as pltpu
from jax.experimental.pallas import tpu_sc as plsc

K = 16
NC = 2
NS = 16
NW = NC * NS
SUB = 128
ROWS_PER_BLK = 8
EB = SUB * ROWS_PER_BLK


def _gate_tc(ea2, We):
    R = ea2.shape[0]
    blk = 200
    assert R % blk == 0

    def body(ea_ref, we_ref, o_ref):
        o_ref[...] = jax.nn.sigmoid(ea_ref[...] * we_ref[0, 0])

    return pl.pallas_call(
        body,
        out_shape=jax.ShapeDtypeStruct(ea2.shape, jnp.float32),
        grid=(R // blk,),
        in_specs=[
            pl.BlockSpec((blk, 128), lambda i: (i, 0)),
            pl.BlockSpec((1, 1), lambda i: (0, 0)),
        ],
        out_specs=pl.BlockSpec((blk, 128), lambda i: (i, 0)),
    )(ea2, We)


def _sim0_tc(x, p, W0, Wp, b0):
    n = x.shape[0]
    blk = 1000
    assert n % blk == 0

    def body(x_ref, p_ref, w0_ref, wp_ref, b_ref, o_ref):
        acc = jnp.dot(x_ref[...], w0_ref[...], preferred_element_type=jnp.float32)
        acc += jnp.dot(p_ref[...], wp_ref[...], preferred_element_type=jnp.float32)
        o_ref[...] = jnp.maximum(acc + b_ref[...], 0.0)

    return pl.pallas_call(
        body,
        out_shape=jax.ShapeDtypeStruct((n, K), jnp.float32),
        grid=(n // blk,),
        in_specs=[
            pl.BlockSpec((blk, x.shape[1]), lambda i: (i, 0)),
            pl.BlockSpec((blk, p.shape[1]), lambda i: (i, 0)),
            pl.BlockSpec(W0.shape, lambda i: (0, 0)),
            pl.BlockSpec(Wp.shape, lambda i: (0, 0)),
            pl.BlockSpec((1, K), lambda i: (0, 0)),
        ],
        out_specs=pl.BlockSpec((blk, K), lambda i: (i, 0)),
    )(x, p, W0, Wp, b0.reshape(1, K))


def _layer_tc(partial, W, b):
    n = partial.shape[1]
    blk = 1000
    assert n % blk == 0

    def body(p_ref, w_ref, b_ref, o_ref):
        h = p_ref[0] + p_ref[1]
        acc = jnp.dot(h, w_ref[...], preferred_element_type=jnp.float32)
        o_ref[...] = jnp.maximum(acc + b_ref[...], 0.0)

    return pl.pallas_call(
        body,
        out_shape=jax.ShapeDtypeStruct((n, K), jnp.float32),
        grid=(n // blk,),
        in_specs=[
            pl.BlockSpec((2, blk, K), lambda i: (0, i, 0)),
            pl.BlockSpec((K, K), lambda i: (0, 0)),
            pl.BlockSpec((1, K), lambda i: (0, 0)),
        ],
        out_specs=pl.BlockSpec((blk, K), lambda i: (i, 0)),
    )(partial, W, b.reshape(1, K))


def _sum_tc(partial):
    n = partial.shape[1]
    blk = 1000

    def body(p_ref, o_ref):
        o_ref[...] = p_ref[0] + p_ref[1]

    return pl.pallas_call(
        body,
        out_shape=jax.ShapeDtypeStruct((n, K), jnp.float32),
        grid=(n // blk,),
        in_specs=[pl.BlockSpec((2, blk, K), lambda i: (0, i, 0))],
        out_specs=pl.BlockSpec((blk, K), lambda i: (i, 0)),
    )(partial)


def _sc_pass(sim, src2, dst2, gate2, bpw):
    n = sim.shape[0]
    stripe = n // NS
    zb = 125
    mesh = plsc.VectorSubcoreMesh(core_axis_name="c", subcore_axis_name="s")

    @functools.partial(
        pl.kernel,
        out_type=jax.ShapeDtypeStruct((NC, n, K), jnp.float32),
        mesh=mesh,
        scratch_types=[
            pltpu.VMEM((ROWS_PER_BLK, SUB), jnp.int32),
            pltpu.VMEM((ROWS_PER_BLK, SUB), jnp.int32),
            pltpu.VMEM((ROWS_PER_BLK, SUB), jnp.float32),
            pltpu.VMEM((SUB, K), jnp.float32),
            pltpu.VMEM_SHARED((n, K), jnp.float32),
            pltpu.SemaphoreType.DMA,
        ],
    )
    def sc_kernel(sim_hbm, src_hbm, dst_hbm, gate_hbm, out_hbm,
                  src_v, dst_v, gate_v, rows_v, acc_sh, sem):
        c = lax.axis_index("c")
        s = lax.axis_index("s")
        wid = s * NC + c

        @pl.loop(0, SUB)
        def _(i):
            rows_v[i, :] = jnp.zeros((K,), jnp.float32)

        @pl.loop(0, stripe // zb)
        def _(t):
            pltpu.sync_copy(rows_v.at[pl.ds(0, zb)],
                            acc_sh.at[pl.ds(s * stripe + t * zb, zb)])

        plsc.subcore_barrier()

        row_base = wid * (bpw * ROWS_PER_BLK)

        @pl.loop(0, bpw)
        def _(kblk):
            r0 = row_base + kblk * ROWS_PER_BLK
            pltpu.sync_copy(src_hbm.at[pl.ds(r0, ROWS_PER_BLK)], src_v)
            pltpu.sync_copy(dst_hbm.at[pl.ds(r0, ROWS_PER_BLK)], dst_v)
            pltpu.sync_copy(gate_hbm.at[pl.ds(r0, ROWS_PER_BLK)], gate_v)
            for j in range(ROWS_PER_BLK):
                pltpu.async_copy(sim_hbm.at[src_v.at[j]], rows_v, sem).wait()

                @pl.loop(0, SUB)
                def _(b):
                    rows_v[b, :] = rows_v[b, :] * gate_v[j, b]

                pltpu.sync_copy(rows_v, acc_sh.at[dst_v.at[j]], add=True)

        plsc.subcore_barrier()

        @pl.loop(0, stripe // zb)
        def _(t):
            off = s * stripe + t * zb
            pltpu.sync_copy(acc_sh.at[pl.ds(off, zb)],
                            out_hbm.at[c].at[pl.ds(off, zb)])

    return sc_kernel(sim, src2, dst2, gate2)


def kernel(x, p, edge_attr, edge_index, W0, Wp, b0, W1, b1, W2, b2, We):
    E = edge_index.shape[1]
    src = edge_index[0]
    dst = edge_index[1]

    gate = _gate_tc(edge_attr.reshape(-1, 128), We).reshape(-1)

    chunk = NW * EB
    Epad = ((E + chunk - 1) // chunk) * chunk
    padn = Epad - E
    src2 = jnp.pad(src, (0, padn)).reshape(-1, SUB)
    dst2 = jnp.pad(dst, (0, padn)).reshape(-1, SUB)
    gate2 = jnp.pad(gate, (0, padn)).reshape(-1, SUB)
    bpw = Epad // (NW * EB)

    sim = _sim0_tc(x, p, W0, Wp, b0)
    partial = _sc_pass(sim, src2, dst2, gate2, bpw)
    sim = _layer_tc(partial, W1, b1)
    partial = _sc_pass(sim, src2, dst2, gate2, bpw)
    sim = _layer_tc(partial, W2, b2)
    partial = _sc_pass(sim, src2, dst2, gate2, bpw)
    return _sum_tc(partial)


# speedup vs baseline: 17.2588x; 17.2588x over previous
import functools

import jax
import jax.numpy as jnp
from jax import lax
from jax.experimental import pallas as pl
from jax.experimental.pallas import tpu as pltpu
from jax.experimental.pallas import tpu_sc as plsc

K = 16
NC = 2
NS = 16
NW = NC * NS
SUB = 128
ROWS_PER_BLK = 8
EB = SUB * ROWS_PER_BLK


def _gate_tc(ea2, We):
    R = ea2.shape[0]
    blk = 200
    assert R % blk == 0

    def body(ea_ref, we_ref, o_ref):
        o_ref[...] = jax.nn.sigmoid(ea_ref[...] * we_ref[0, 0])

    return pl.pallas_call(
        body,
        out_shape=jax.ShapeDtypeStruct(ea2.shape, jnp.float32),
        grid=(R // blk,),
        in_specs=[
            pl.BlockSpec((blk, 128), lambda i: (i, 0)),
            pl.BlockSpec((1, 1), lambda i: (0, 0)),
        ],
        out_specs=pl.BlockSpec((blk, 128), lambda i: (i, 0)),
    )(ea2, We)


def _sim0_tc(x, p, W0, Wp, b0):
    n = x.shape[0]
    blk = 1000
    assert n % blk == 0

    def body(x_ref, p_ref, w0_ref, wp_ref, b_ref, o_ref):
        acc = jnp.dot(x_ref[...], w0_ref[...], preferred_element_type=jnp.float32)
        acc += jnp.dot(p_ref[...], wp_ref[...], preferred_element_type=jnp.float32)
        o_ref[...] = jnp.maximum(acc + b_ref[...], 0.0)

    return pl.pallas_call(
        body,
        out_shape=jax.ShapeDtypeStruct((n, K), jnp.float32),
        grid=(n // blk,),
        in_specs=[
            pl.BlockSpec((blk, x.shape[1]), lambda i: (i, 0)),
            pl.BlockSpec((blk, p.shape[1]), lambda i: (i, 0)),
            pl.BlockSpec(W0.shape, lambda i: (0, 0)),
            pl.BlockSpec(Wp.shape, lambda i: (0, 0)),
            pl.BlockSpec((1, K), lambda i: (0, 0)),
        ],
        out_specs=pl.BlockSpec((blk, K), lambda i: (i, 0)),
    )(x, p, W0, Wp, b0.reshape(1, K))


def _layer_tc(partial, W, b):
    n = partial.shape[1]
    blk = 1000
    assert n % blk == 0

    def body(p_ref, w_ref, b_ref, o_ref):
        h = p_ref[0] + p_ref[1]
        acc = jnp.dot(h, w_ref[...], preferred_element_type=jnp.float32)
        o_ref[...] = jnp.maximum(acc + b_ref[...], 0.0)

    return pl.pallas_call(
        body,
        out_shape=jax.ShapeDtypeStruct((n, K), jnp.float32),
        grid=(n // blk,),
        in_specs=[
            pl.BlockSpec((2, blk, K), lambda i: (0, i, 0)),
            pl.BlockSpec((K, K), lambda i: (0, 0)),
            pl.BlockSpec((1, K), lambda i: (0, 0)),
        ],
        out_specs=pl.BlockSpec((blk, K), lambda i: (i, 0)),
    )(partial, W, b.reshape(1, K))


def _sum_tc(partial):
    n = partial.shape[1]
    blk = 1000

    def body(p_ref, o_ref):
        o_ref[...] = p_ref[0] + p_ref[1]

    return pl.pallas_call(
        body,
        out_shape=jax.ShapeDtypeStruct((n, K), jnp.float32),
        grid=(n // blk,),
        in_specs=[pl.BlockSpec((2, blk, K), lambda i: (0, i, 0))],
        out_specs=pl.BlockSpec((blk, K), lambda i: (i, 0)),
    )(partial)


def _sc_pass(sim, src2, dst2, gate2, bpw):
    n = sim.shape[0]
    zrows = 160
    nchunks = n // zrows
    chunks_per_sub = (nchunks + NS - 1) // NS
    mesh = plsc.VectorSubcoreMesh(core_axis_name="c", subcore_axis_name="s")

    @functools.partial(
        pl.kernel,
        out_type=jax.ShapeDtypeStruct((NC, n, K), jnp.float32),
        mesh=mesh,
        scratch_types=[
            pltpu.VMEM((ROWS_PER_BLK, SUB), jnp.int32),
            pltpu.VMEM((ROWS_PER_BLK, SUB), jnp.int32),
            pltpu.VMEM((ROWS_PER_BLK, SUB), jnp.float32),
            pltpu.VMEM((SUB, K), jnp.float32),
            pltpu.VMEM((zrows, K), jnp.float32),
            pltpu.VMEM_SHARED((n, K), jnp.float32),
            pltpu.SemaphoreType.DMA,
        ],
        compiler_params=pltpu.CompilerParams(use_tc_tiling_on_sc=False),
    )
    def sc_kernel(sim_hbm, src_hbm, dst_hbm, gate_hbm, out_hbm,
                  src_v, dst_v, gate_v, rows_v, zbuf_v, acc_sh, sem):
        c = lax.axis_index("c")
        s = lax.axis_index("s")
        wid = s * NC + c

        @pl.loop(0, zrows)
        def _(i):
            zbuf_v[i, :] = jnp.zeros((K,), jnp.float32)

        @pl.loop(0, chunks_per_sub)
        def _(t):
            cidx = t * NS + s

            @pl.when(cidx < nchunks)
            def _():
                pltpu.sync_copy(zbuf_v, acc_sh.at[pl.ds(cidx * zrows, zrows)])

        plsc.subcore_barrier()

        row_base = wid * (bpw * ROWS_PER_BLK)

        @pl.loop(0, bpw)
        def _(kblk):
            r0 = row_base + kblk * ROWS_PER_BLK
            pltpu.sync_copy(src_hbm.at[pl.ds(r0, ROWS_PER_BLK)], src_v)
            pltpu.sync_copy(dst_hbm.at[pl.ds(r0, ROWS_PER_BLK)], dst_v)
            pltpu.sync_copy(gate_hbm.at[pl.ds(r0, ROWS_PER_BLK)], gate_v)
            for j in range(ROWS_PER_BLK):
                pltpu.async_copy(sim_hbm.at[src_v.at[j]], rows_v, sem).wait()

                @pl.loop(0, SUB // K)
                def _(q):
                    gv = gate_v[j, pl.ds(q * K, K)]
                    for i in range(K):
                        b = q * K + i
                        rows_v[b, :] = rows_v[b, :] * gv[i]

                pltpu.sync_copy(rows_v, acc_sh.at[dst_v.at[j]], add=True)

        plsc.subcore_barrier()

        @pl.loop(0, chunks_per_sub)
        def _(t):
            cidx = t * NS + s

            @pl.when(cidx < nchunks)
            def _():
                off = cidx * zrows
                pltpu.sync_copy(acc_sh.at[pl.ds(off, zrows)],
                                out_hbm.at[c].at[pl.ds(off, zrows)])

    return sc_kernel(sim, src2, dst2, gate2)


def kernel(x, p, edge_attr, edge_index, W0, Wp, b0, W1, b1, W2, b2, We):
    E = edge_index.shape[1]
    src = edge_index[0]
    dst = edge_index[1]

    gate = _gate_tc(edge_attr.reshape(-1, 128), We).reshape(-1)

    chunk = NW * EB
    Epad = ((E + chunk - 1) // chunk) * chunk
    padn = Epad - E
    src2 = jnp.pad(src, (0, padn)).reshape(-1, SUB)
    dst2 = jnp.pad(dst, (0, padn)).reshape(-1, SUB)
    gate2 = jnp.pad(gate, (0, padn)).reshape(-1, SUB)
    bpw = Epad // (NW * EB)

    sim = _sim0_tc(x, p, W0, Wp, b0)
    partial = _sc_pass(sim, src2, dst2, gate2, bpw)
    sim = _layer_tc(partial, W1, b1)
    partial = _sc_pass(sim, src2, dst2, gate2, bpw)
    sim = _layer_tc(partial, W2, b2)
    partial = _sc_pass(sim, src2, dst2, gate2, bpw)
    return _sum_tc(partial)


# speedup vs baseline: 33.7056x; 1.9529x over previous
import functools

import jax
import jax.numpy as jnp
from jax import lax
from jax.experimental import pallas as pl
from jax.experimental.pallas import tpu as pltpu
from jax.experimental.pallas import tpu_sc as plsc

K = 16
NC = 2
NS = 16
NW = NC * NS
SUB = 128
ROWS_PER_BLK = 4
EB = SUB * ROWS_PER_BLK


def _gate_tc(ea2, We):
    R = ea2.shape[0]
    blk = 200
    assert R % blk == 0

    def body(ea_ref, we_ref, o_ref):
        o_ref[...] = jax.nn.sigmoid(ea_ref[...] * we_ref[0, 0])

    return pl.pallas_call(
        body,
        out_shape=jax.ShapeDtypeStruct(ea2.shape, jnp.float32),
        grid=(R // blk,),
        in_specs=[
            pl.BlockSpec((blk, 128), lambda i: (i, 0)),
            pl.BlockSpec((1, 1), lambda i: (0, 0)),
        ],
        out_specs=pl.BlockSpec((blk, 128), lambda i: (i, 0)),
    )(ea2, We)


def _sim0_tc(x, p, W0, Wp, b0):
    n = x.shape[0]
    blk = 1000
    assert n % blk == 0

    def body(x_ref, p_ref, w0_ref, wp_ref, b_ref, o_ref):
        acc = jnp.dot(x_ref[...], w0_ref[...], preferred_element_type=jnp.float32)
        acc += jnp.dot(p_ref[...], wp_ref[...], preferred_element_type=jnp.float32)
        o_ref[...] = jnp.maximum(acc + b_ref[...], 0.0)

    return pl.pallas_call(
        body,
        out_shape=jax.ShapeDtypeStruct((n, K), jnp.float32),
        grid=(n // blk,),
        in_specs=[
            pl.BlockSpec((blk, x.shape[1]), lambda i: (i, 0)),
            pl.BlockSpec((blk, p.shape[1]), lambda i: (i, 0)),
            pl.BlockSpec(W0.shape, lambda i: (0, 0)),
            pl.BlockSpec(Wp.shape, lambda i: (0, 0)),
            pl.BlockSpec((1, K), lambda i: (0, 0)),
        ],
        out_specs=pl.BlockSpec((blk, K), lambda i: (i, 0)),
    )(x, p, W0, Wp, b0.reshape(1, K))


def _layer_tc(partial, W, b):
    n = partial.shape[1]
    blk = 1000
    assert n % blk == 0

    def body(p_ref, w_ref, b_ref, o_ref):
        h = p_ref[0] + p_ref[1]
        acc = jnp.dot(h, w_ref[...], preferred_element_type=jnp.float32)
        o_ref[...] = jnp.maximum(acc + b_ref[...], 0.0)

    return pl.pallas_call(
        body,
        out_shape=jax.ShapeDtypeStruct((n, K), jnp.float32),
        grid=(n // blk,),
        in_specs=[
            pl.BlockSpec((2, blk, K), lambda i: (0, i, 0)),
            pl.BlockSpec((K, K), lambda i: (0, 0)),
            pl.BlockSpec((1, K), lambda i: (0, 0)),
        ],
        out_specs=pl.BlockSpec((blk, K), lambda i: (i, 0)),
    )(partial, W, b.reshape(1, K))


def _sum_tc(partial):
    n = partial.shape[1]
    blk = 1000

    def body(p_ref, o_ref):
        o_ref[...] = p_ref[0] + p_ref[1]

    return pl.pallas_call(
        body,
        out_shape=jax.ShapeDtypeStruct((n, K), jnp.float32),
        grid=(n // blk,),
        in_specs=[pl.BlockSpec((2, blk, K), lambda i: (0, i, 0))],
        out_specs=pl.BlockSpec((blk, K), lambda i: (i, 0)),
    )(partial)


def _sc_pass(sim, src2, dst2, gate2, bpw):
    n = sim.shape[0]
    zrows = 160
    nchunks = n // zrows
    chunks_per_sub = (nchunks + NS - 1) // NS
    mesh = plsc.VectorSubcoreMesh(core_axis_name="c", subcore_axis_name="s")

    @functools.partial(
        pl.kernel,
        out_type=jax.ShapeDtypeStruct((NC, n, K), jnp.float32),
        mesh=mesh,
        scratch_types=[
            pltpu.VMEM((2, ROWS_PER_BLK, SUB), jnp.int32),
            pltpu.VMEM((2, ROWS_PER_BLK, SUB), jnp.int32),
            pltpu.VMEM((2, ROWS_PER_BLK, SUB), jnp.float32),
            pltpu.VMEM((2, ROWS_PER_BLK, SUB, K), jnp.float32),
            pltpu.VMEM((zrows, K), jnp.float32),
            pltpu.VMEM_SHARED((n, K), jnp.float32),
            pltpu.SemaphoreType.DMA,
            pltpu.SemaphoreType.DMA,
            pltpu.SemaphoreType.DMA,
            pltpu.SemaphoreType.DMA,
        ],
        compiler_params=pltpu.CompilerParams(use_tc_tiling_on_sc=False),
    )
    def sc_kernel(sim_hbm, src_hbm, dst_hbm, gate_hbm, out_hbm,
                  src_v, dst_v, gate_v, rows_v, zbuf_v, acc_sh,
                  gsem0, gsem1, scsem, stsem):
        c = lax.axis_index("c")
        s = lax.axis_index("s")
        wid = s * NC + c
        gsems = (gsem0, gsem1)

        @pl.loop(0, zrows)
        def _(i):
            zbuf_v[i, :] = jnp.zeros((K,), jnp.float32)

        @pl.loop(0, chunks_per_sub)
        def _(t):
            cidx = t * NS + s

            @pl.when(cidx < nchunks)
            def _():
                pltpu.sync_copy(zbuf_v, acc_sh.at[pl.ds(cidx * zrows, zrows)])

        plsc.subcore_barrier()

        row_base = wid * (bpw * ROWS_PER_BLK)

        def stage(kblk, par, issue):
            r0 = row_base + kblk * ROWS_PER_BLK
            op = pltpu.async_copy if issue else pltpu.make_async_copy
            cps = [
                op(src_hbm.at[pl.ds(r0, ROWS_PER_BLK)], src_v.at[par], stsem),
                op(dst_hbm.at[pl.ds(r0, ROWS_PER_BLK)], dst_v.at[par], stsem),
                op(gate_hbm.at[pl.ds(r0, ROWS_PER_BLK)], gate_v.at[par], stsem),
            ]
            if not issue:
                for cp in cps:
                    cp.wait()

        def gathers(par, issue):
            op = pltpu.async_copy if issue else pltpu.make_async_copy
            for j in range(ROWS_PER_BLK):
                cp = op(sim_hbm.at[src_v.at[par].at[j]],
                        rows_v.at[par].at[j], gsems[par])
                if not issue:
                    cp.wait()

        def process(kblk, t, par, other):
            gathers(par, issue=False)

            def prefetch():
                stage(kblk + 1, other, issue=False)
                gathers(other, issue=True)

            if par == 0:
                prefetch()
            else:
                pl.when(t < bpw // 2 - 1)(prefetch)

            @pl.loop(0, ROWS_PER_BLK)
            def _(j):
                @pl.loop(0, SUB // K)
                def _(q):
                    gv = gate_v[par, j, pl.ds(q * K, K)]
                    for i in range(K):
                        b = q * K + i
                        rows_v[par, j, b, :] = rows_v[par, j, b, :] * gv[i]

            cps = [pltpu.async_copy(rows_v.at[par].at[j],
                                    acc_sh.at[dst_v.at[par].at[j]],
                                    scsem, add=True)
                   for j in range(ROWS_PER_BLK)]
            for cp in cps:
                cp.wait()

            pl.when(t < bpw // 2 - 1)(lambda: stage(kblk + 2, par, issue=True))

        r0 = row_base
        pltpu.sync_copy(src_hbm.at[pl.ds(r0, ROWS_PER_BLK)], src_v.at[0])
        pltpu.sync_copy(dst_hbm.at[pl.ds(r0, ROWS_PER_BLK)], dst_v.at[0])
        pltpu.sync_copy(gate_hbm.at[pl.ds(r0, ROWS_PER_BLK)], gate_v.at[0])
        gathers(0, issue=True)
        stage(1, 1, issue=True)

        @pl.loop(0, bpw // 2)
        def _(t):
            process(2 * t, t, 0, 1)
            process(2 * t + 1, t, 1, 0)

        plsc.subcore_barrier()

        @pl.loop(0, chunks_per_sub)
        def _(t):
            cidx = t * NS + s

            @pl.when(cidx < nchunks)
            def _():
                off = cidx * zrows
                pltpu.sync_copy(acc_sh.at[pl.ds(off, zrows)],
                                out_hbm.at[c].at[pl.ds(off, zrows)])

    return sc_kernel(sim, src2, dst2, gate2)


def kernel(x, p, edge_attr, edge_index, W0, Wp, b0, W1, b1, W2, b2, We):
    E = edge_index.shape[1]
    src = edge_index[0]
    dst = edge_index[1]

    gate = _gate_tc(edge_attr.reshape(-1, 128), We).reshape(-1)

    chunk = NW * EB
    Epad = ((E + chunk - 1) // chunk) * chunk
    padn = Epad - E
    src2 = jnp.pad(src, (0, padn)).reshape(-1, SUB)
    dst2 = jnp.pad(dst, (0, padn)).reshape(-1, SUB)
    gate2 = jnp.pad(gate, (0, padn)).reshape(-1, SUB)
    bpw = Epad // (NW * EB)

    sim = _sim0_tc(x, p, W0, Wp, b0)
    partial = _sc_pass(sim, src2, dst2, gate2, bpw)
    sim = _layer_tc(partial, W1, b1)
    partial = _sc_pass(sim, src2, dst2, gate2, bpw)
    sim = _layer_tc(partial, W2, b2)
    partial = _sc_pass(sim, src2, dst2, gate2, bpw)
    return _sum_tc(partial)


# speedup vs baseline: 39.1646x; 1.1620x over previous
import functools

import jax
import jax.numpy as jnp
from jax import lax
from jax.experimental import pallas as pl
from jax.experimental.pallas import tpu as pltpu
from jax.experimental.pallas import tpu_sc as plsc

K = 16
NC = 2
NS = 16
NW = NC * NS
SUB = 128
ROWS_PER_BLK = 4
EB = SUB * ROWS_PER_BLK


def _gate_tc(ea2, We):
    R = ea2.shape[0]
    blk = 200
    assert R % blk == 0

    def body(ea_ref, we_ref, o_ref):
        o_ref[...] = jax.nn.sigmoid(ea_ref[...] * we_ref[0, 0])

    return pl.pallas_call(
        body,
        out_shape=jax.ShapeDtypeStruct(ea2.shape, jnp.float32),
        grid=(R // blk,),
        in_specs=[
            pl.BlockSpec((blk, 128), lambda i: (i, 0)),
            pl.BlockSpec((1, 1), lambda i: (0, 0)),
        ],
        out_specs=pl.BlockSpec((blk, 128), lambda i: (i, 0)),
    )(ea2, We)


def _sim0_tc(x, p, W0, Wp, b0):
    n = x.shape[0]
    blk = 1000
    assert n % blk == 0

    def body(x_ref, p_ref, w0_ref, wp_ref, b_ref, o_ref):
        acc = jnp.dot(x_ref[...], w0_ref[...], preferred_element_type=jnp.float32)
        acc += jnp.dot(p_ref[...], wp_ref[...], preferred_element_type=jnp.float32)
        o_ref[...] = jnp.maximum(acc + b_ref[...], 0.0)

    return pl.pallas_call(
        body,
        out_shape=jax.ShapeDtypeStruct((n, K), jnp.float32),
        grid=(n // blk,),
        in_specs=[
            pl.BlockSpec((blk, x.shape[1]), lambda i: (i, 0)),
            pl.BlockSpec((blk, p.shape[1]), lambda i: (i, 0)),
            pl.BlockSpec(W0.shape, lambda i: (0, 0)),
            pl.BlockSpec(Wp.shape, lambda i: (0, 0)),
            pl.BlockSpec((1, K), lambda i: (0, 0)),
        ],
        out_specs=pl.BlockSpec((blk, K), lambda i: (i, 0)),
    )(x, p, W0, Wp, b0.reshape(1, K))


def _layer_tc(partial, W, b):
    n = partial.shape[1]
    blk = 1000
    assert n % blk == 0

    def body(p_ref, w_ref, b_ref, o_ref):
        h = p_ref[0] + p_ref[1]
        acc = jnp.dot(h, w_ref[...], preferred_element_type=jnp.float32)
        o_ref[...] = jnp.maximum(acc + b_ref[...], 0.0)

    return pl.pallas_call(
        body,
        out_shape=jax.ShapeDtypeStruct((n, K), jnp.float32),
        grid=(n // blk,),
        in_specs=[
            pl.BlockSpec((2, blk, K), lambda i: (0, i, 0)),
            pl.BlockSpec((K, K), lambda i: (0, 0)),
            pl.BlockSpec((1, K), lambda i: (0, 0)),
        ],
        out_specs=pl.BlockSpec((blk, K), lambda i: (i, 0)),
    )(partial, W, b.reshape(1, K))


def _sum_tc(partial):
    n = partial.shape[1]
    blk = 1000

    def body(p_ref, o_ref):
        o_ref[...] = p_ref[0] + p_ref[1]

    return pl.pallas_call(
        body,
        out_shape=jax.ShapeDtypeStruct((n, K), jnp.float32),
        grid=(n // blk,),
        in_specs=[pl.BlockSpec((2, blk, K), lambda i: (0, i, 0))],
        out_specs=pl.BlockSpec((blk, K), lambda i: (i, 0)),
    )(partial)


def _sc_pass(sim, src2, dst2, gate2):
    n = sim.shape[0]
    zrows = 160
    nchunks = n // zrows
    chunks_per_sub = (nchunks + NS - 1) // NS
    R = src2.shape[0]
    pair_rows = 2 * ROWS_PER_BLK
    rpw = -(-(R // NW) // pair_rows) * pair_rows
    last = R - (NW - 1) * rpw
    assert 0 < last <= rpw and last % pair_rows == 0
    full_pairs = rpw // pair_rows
    last_pairs = last // pair_rows
    mesh = plsc.VectorSubcoreMesh(core_axis_name="c", subcore_axis_name="s")

    @functools.partial(
        pl.kernel,
        out_type=jax.ShapeDtypeStruct((NC, n, K), jnp.float32),
        mesh=mesh,
        scratch_types=[
            pltpu.VMEM((2, ROWS_PER_BLK, SUB), jnp.int32),
            pltpu.VMEM((2, ROWS_PER_BLK, SUB), jnp.int32),
            pltpu.VMEM((2, ROWS_PER_BLK, SUB), jnp.float32),
            pltpu.VMEM((2, ROWS_PER_BLK, SUB, K), jnp.float32),
            pltpu.VMEM((zrows, K), jnp.float32),
            pltpu.VMEM_SHARED((n, K), jnp.float32),
            pltpu.SemaphoreType.DMA,
            pltpu.SemaphoreType.DMA,
            pltpu.SemaphoreType.DMA,
            pltpu.SemaphoreType.DMA,
        ],
        compiler_params=pltpu.CompilerParams(use_tc_tiling_on_sc=False),
    )
    def sc_kernel(sim_hbm, src_hbm, dst_hbm, gate_hbm, out_hbm,
                  src_v, dst_v, gate_v, rows_v, zbuf_v, acc_sh,
                  gsem0, gsem1, scsem, stsem):
        c = lax.axis_index("c")
        s = lax.axis_index("s")
        wid = s * NC + c
        gsems = (gsem0, gsem1)

        @pl.loop(0, zrows)
        def _(i):
            zbuf_v[i, :] = jnp.zeros((K,), jnp.float32)

        @pl.loop(0, chunks_per_sub)
        def _(t):
            cidx = t * NS + s

            @pl.when(cidx < nchunks)
            def _():
                pltpu.sync_copy(zbuf_v, acc_sh.at[pl.ds(cidx * zrows, zrows)])

        plsc.subcore_barrier()

        row_base = wid * rpw
        npairs = jnp.where(wid == NW - 1, last_pairs, full_pairs)

        def blk_r0(kblk):
            return row_base + kblk * ROWS_PER_BLK

        def stage_sg(kblk, par, issue):
            r0 = blk_r0(kblk)
            op = pltpu.async_copy if issue else pltpu.make_async_copy
            cps = [
                op(src_hbm.at[pl.ds(r0, ROWS_PER_BLK)], src_v.at[par], stsem),
                op(gate_hbm.at[pl.ds(r0, ROWS_PER_BLK)], gate_v.at[par], stsem),
            ]
            if not issue:
                for cp in cps:
                    cp.wait()

        def stage_d(kblk, par, issue):
            r0 = blk_r0(kblk)
            op = pltpu.async_copy if issue else pltpu.make_async_copy
            cp = op(dst_hbm.at[pl.ds(r0, ROWS_PER_BLK)], dst_v.at[par], stsem)
            if not issue:
                cp.wait()

        def gathers(par, issue):
            op = pltpu.async_copy if issue else pltpu.make_async_copy
            for j in range(ROWS_PER_BLK):
                cp = op(sim_hbm.at[src_v.at[par].at[j]],
                        rows_v.at[par].at[j], gsems[par])
                if not issue:
                    cp.wait()

        def scatters(par, issue):
            for j in range(ROWS_PER_BLK):
                if issue:
                    pltpu.async_copy(rows_v.at[par].at[j],
                                     acc_sh.at[dst_v.at[par].at[j]],
                                     scsem, add=True)
                else:
                    pltpu.make_async_copy(rows_v.at[par].at[j],
                                          acc_sh.at[dst_v.at[par].at[j]],
                                          scsem).wait()

        def maybe(cond, fn):
            if cond is True:
                fn()
            else:
                pl.when(cond)(fn)

        def process(kblk, t, par, other):
            has_next = t < npairs - 1 if par else True
            has_next2 = t < npairs - 1
            not_first = True if par else t > 0

            gathers(par, issue=False)
            maybe(has_next, lambda: stage_sg(kblk + 1, other, issue=False))
            maybe(not_first, lambda: stage_d(kblk, par, issue=False))
            maybe(not_first, lambda: scatters(other, issue=False))
            maybe(has_next, lambda: stage_d(kblk + 1, other, issue=True))
            maybe(has_next, lambda: gathers(other, issue=True))

            @pl.loop(0, ROWS_PER_BLK)
            def _(j):
                @pl.loop(0, SUB // K)
                def _(q):
                    gv = gate_v[par, j, pl.ds(q * K, K)]
                    for i in range(K):
                        b = q * K + i
                        rows_v[par, j, b, :] = rows_v[par, j, b, :] * gv[i]

            scatters(par, issue=True)
            maybe(has_next2, lambda: stage_sg(kblk + 2, par, issue=True))

        r0 = row_base
        pltpu.sync_copy(src_hbm.at[pl.ds(r0, ROWS_PER_BLK)], src_v.at[0])
        pltpu.sync_copy(dst_hbm.at[pl.ds(r0, ROWS_PER_BLK)], dst_v.at[0])
        pltpu.sync_copy(gate_hbm.at[pl.ds(r0, ROWS_PER_BLK)], gate_v.at[0])
        gathers(0, issue=True)
        stage_sg(1, 1, issue=True)

        @pl.loop(0, npairs)
        def _(t):
            process(2 * t, t, 0, 1)
            process(2 * t + 1, t, 1, 0)

        scatters(1, issue=False)
        plsc.subcore_barrier()

        @pl.loop(0, chunks_per_sub)
        def _(t):
            cidx = t * NS + s

            @pl.when(cidx < nchunks)
            def _():
                off = cidx * zrows
                pltpu.sync_copy(acc_sh.at[pl.ds(off, zrows)],
                                out_hbm.at[c].at[pl.ds(off, zrows)])

    return sc_kernel(sim, src2, dst2, gate2)


def kernel(x, p, edge_attr, edge_index, W0, Wp, b0, W1, b1, W2, b2, We):
    E = edge_index.shape[1]
    src2 = edge_index[0].reshape(-1, SUB)
    dst2 = edge_index[1].reshape(-1, SUB)
    gate2 = _gate_tc(edge_attr.reshape(-1, SUB), We)

    sim = _sim0_tc(x, p, W0, Wp, b0)
    partial = _sc_pass(sim, src2, dst2, gate2)
    sim = _layer_tc(partial, W1, b1)
    partial = _sc_pass(sim, src2, dst2, gate2)
    sim = _layer_tc(partial, W2, b2)
    partial = _sc_pass(sim, src2, dst2, gate2)
    return _sum_tc(partial)


# speedup vs baseline: 53.2730x; 1.3602x over previous
import functools

import jax
import jax.numpy as jnp
from jax import lax
from jax.experimental import pallas as pl
from jax.experimental.pallas import tpu as pltpu
from jax.experimental.pallas import tpu_sc as plsc

K = 16
NC = 2
NS = 16
NW = NC * NS
SUB = 128
ROWS_PER_BLK = 4
EB = SUB * ROWS_PER_BLK


def _gate_tc(ea2, We):
    R = ea2.shape[0]
    blk = 200
    assert R % blk == 0

    def body(ea_ref, we_ref, o_ref):
        o_ref[...] = jax.nn.sigmoid(ea_ref[...] * we_ref[0, 0])

    return pl.pallas_call(
        body,
        out_shape=jax.ShapeDtypeStruct(ea2.shape, jnp.float32),
        grid=(R // blk,),
        in_specs=[
            pl.BlockSpec((blk, 128), lambda i: (i, 0)),
            pl.BlockSpec((1, 1), lambda i: (0, 0)),
        ],
        out_specs=pl.BlockSpec((blk, 128), lambda i: (i, 0)),
    )(ea2, We)


PACK = 128 // K
_G = 25
_BR = 500


def _sim0_tc(x, p, W0, Wp, b0):
    n = x.shape[0]
    dx, dp = x.shape[1], p.shape[1]
    assert n == _G * _BR * PACK
    x4 = x.reshape(_G, _BR, PACK * dx)
    p4 = p.reshape(_G, _BR, PACK * dp)
    eye = jnp.eye(PACK, dtype=jnp.float32)
    BWx = jnp.kron(eye, W0)
    BWp = jnp.kron(eye, Wp)
    bt = jnp.tile(b0, PACK).reshape(1, PACK * K)

    def body(x_ref, p_ref, wx_ref, wp_ref, b_ref, o_ref):
        acc = jnp.dot(x_ref[0], wx_ref[...], preferred_element_type=jnp.float32)
        acc += jnp.dot(p_ref[0], wp_ref[...], preferred_element_type=jnp.float32)
        o_ref[0] = jnp.maximum(acc + b_ref[...], 0.0)

    out = pl.pallas_call(
        body,
        out_shape=jax.ShapeDtypeStruct((_G, _BR, PACK * K), jnp.float32),
        grid=(_G,),
        in_specs=[
            pl.BlockSpec((1, _BR, PACK * dx), lambda i: (i, 0, 0)),
            pl.BlockSpec((1, _BR, PACK * dp), lambda i: (i, 0, 0)),
            pl.BlockSpec(BWx.shape, lambda i: (0, 0)),
            pl.BlockSpec(BWp.shape, lambda i: (0, 0)),
            pl.BlockSpec((1, PACK * K), lambda i: (0, 0)),
        ],
        out_specs=pl.BlockSpec((1, _BR, PACK * K), lambda i: (i, 0, 0)),
    )(x4, p4, BWx, BWp, bt)
    return out.reshape(n, K)


def _layer_tc(partial, W, b):
    n = partial.shape[1]
    assert n == _G * _BR * PACK
    p4 = partial.reshape(2, _G, _BR, PACK * K)
    BW = jnp.kron(jnp.eye(PACK, dtype=jnp.float32), W)
    bt = jnp.tile(b, PACK).reshape(1, PACK * K)

    def body(p_ref, w_ref, b_ref, o_ref):
        h = p_ref[0, 0] + p_ref[1, 0]
        acc = jnp.dot(h, w_ref[...], preferred_element_type=jnp.float32)
        o_ref[0] = jnp.maximum(acc + b_ref[...], 0.0)

    out = pl.pallas_call(
        body,
        out_shape=jax.ShapeDtypeStruct((_G, _BR, PACK * K), jnp.float32),
        grid=(_G,),
        in_specs=[
            pl.BlockSpec((2, 1, _BR, PACK * K), lambda i: (0, i, 0, 0)),
            pl.BlockSpec(BW.shape, lambda i: (0, 0)),
            pl.BlockSpec((1, PACK * K), lambda i: (0, 0)),
        ],
        out_specs=pl.BlockSpec((1, _BR, PACK * K), lambda i: (i, 0, 0)),
    )(p4, BW, bt)
    return out.reshape(n, K)


def _sum_tc(partial):
    n = partial.shape[1]
    p4 = partial.reshape(2, _G, _BR, PACK * K)

    def body(p_ref, o_ref):
        o_ref[0] = p_ref[0, 0] + p_ref[1, 0]

    out = pl.pallas_call(
        body,
        out_shape=jax.ShapeDtypeStruct((_G, _BR, PACK * K), jnp.float32),
        grid=(_G,),
        in_specs=[pl.BlockSpec((2, 1, _BR, PACK * K), lambda i: (0, i, 0, 0))],
        out_specs=pl.BlockSpec((1, _BR, PACK * K), lambda i: (i, 0, 0)),
    )(p4)
    return out.reshape(n, K)


def _sc_pass(sim, ei3, gate2):
    n = sim.shape[0]
    zrows = 160
    nchunks = n // zrows
    chunks_per_sub = (nchunks + NS - 1) // NS
    R = gate2.shape[0]
    pair_rows = 2 * ROWS_PER_BLK
    rpw = -(-(R // NW) // pair_rows) * pair_rows
    last = R - (NW - 1) * rpw
    assert 0 < last <= rpw and last % pair_rows == 0
    full_pairs = rpw // pair_rows
    last_pairs = last // pair_rows
    mesh = plsc.VectorSubcoreMesh(core_axis_name="c", subcore_axis_name="s")

    @functools.partial(
        pl.kernel,
        out_type=jax.ShapeDtypeStruct((NC, n, K), jnp.float32),
        mesh=mesh,
        scratch_types=[
            pltpu.VMEM((2, ROWS_PER_BLK, SUB), jnp.int32),
            pltpu.VMEM((2, ROWS_PER_BLK, SUB), jnp.int32),
            pltpu.VMEM((2, ROWS_PER_BLK, SUB), jnp.float32),
            pltpu.VMEM((2, ROWS_PER_BLK, SUB, K), jnp.float32),
            pltpu.VMEM((zrows, K), jnp.float32),
            pltpu.VMEM_SHARED((n, K), jnp.float32),
            pltpu.SemaphoreType.DMA,
            pltpu.SemaphoreType.DMA,
            pltpu.SemaphoreType.DMA,
            pltpu.SemaphoreType.DMA,
        ],
        compiler_params=pltpu.CompilerParams(use_tc_tiling_on_sc=False),
    )
    def sc_kernel(sim_hbm, ei_hbm, gate_hbm, out_hbm,
                  src_v, dst_v, gate_v, rows_v, zbuf_v, acc_sh,
                  gsem0, gsem1, scsem, stsem):
        c = lax.axis_index("c")
        s = lax.axis_index("s")
        wid = s * NC + c
        gsems = (gsem0, gsem1)

        @pl.loop(0, zrows)
        def _(i):
            zbuf_v[i, :] = jnp.zeros((K,), jnp.float32)

        @pl.loop(0, chunks_per_sub)
        def _(t):
            cidx = t * NS + s

            @pl.when(cidx < nchunks)
            def _():
                pltpu.sync_copy(zbuf_v, acc_sh.at[pl.ds(cidx * zrows, zrows)])

        plsc.subcore_barrier()

        row_base = wid * rpw
        npairs = jnp.where(wid == NW - 1, last_pairs, full_pairs)

        def blk_r0(kblk):
            return row_base + kblk * ROWS_PER_BLK

        def stage_sg(kblk, par, issue):
            r0 = blk_r0(kblk)
            op = pltpu.async_copy if issue else pltpu.make_async_copy
            cps = [
                op(ei_hbm.at[0].at[pl.ds(r0, ROWS_PER_BLK)], src_v.at[par], stsem),
                op(gate_hbm.at[pl.ds(r0, ROWS_PER_BLK)], gate_v.at[par], stsem),
            ]
            if not issue:
                for cp in cps:
                    cp.wait()

        def stage_d(kblk, par, issue):
            r0 = blk_r0(kblk)
            op = pltpu.async_copy if issue else pltpu.make_async_copy
            cp = op(ei_hbm.at[1].at[pl.ds(r0, ROWS_PER_BLK)], dst_v.at[par], stsem)
            if not issue:
                cp.wait()

        def gathers(par, issue):
            op = pltpu.async_copy if issue else pltpu.make_async_copy
            for j in range(ROWS_PER_BLK):
                cp = op(sim_hbm.at[src_v.at[par].at[j]],
                        rows_v.at[par].at[j], gsems[par])
                if not issue:
                    cp.wait()

        def scatters(par, issue):
            for j in range(ROWS_PER_BLK):
                if issue:
                    pltpu.async_copy(rows_v.at[par].at[j],
                                     acc_sh.at[dst_v.at[par].at[j]],
                                     scsem, add=True)
                else:
                    pltpu.make_async_copy(rows_v.at[par].at[j],
                                          acc_sh.at[dst_v.at[par].at[j]],
                                          scsem).wait()

        def maybe(cond, fn):
            if cond is True:
                fn()
            else:
                pl.when(cond)(fn)

        def process(kblk, t, par, other):
            has_next = t < npairs - 1 if par else True
            has_next2 = t < npairs - 1
            not_first = True if par else t > 0

            gathers(par, issue=False)
            maybe(has_next, lambda: stage_sg(kblk + 1, other, issue=False))
            maybe(not_first, lambda: stage_d(kblk, par, issue=False))
            maybe(not_first, lambda: scatters(other, issue=False))
            maybe(has_next, lambda: stage_d(kblk + 1, other, issue=True))
            maybe(has_next, lambda: gathers(other, issue=True))

            @pl.loop(0, ROWS_PER_BLK)
            def _(j):
                @pl.loop(0, SUB // K)
                def _(q):
                    gv = gate_v[par, j, pl.ds(q * K, K)]
                    for i in range(K):
                        b = q * K + i
                        rows_v[par, j, b, :] = rows_v[par, j, b, :] * gv[i]

            scatters(par, issue=True)
            maybe(has_next2, lambda: stage_sg(kblk + 2, par, issue=True))

        r0 = row_base
        pltpu.sync_copy(ei_hbm.at[0].at[pl.ds(r0, ROWS_PER_BLK)], src_v.at[0])
        pltpu.sync_copy(ei_hbm.at[1].at[pl.ds(r0, ROWS_PER_BLK)], dst_v.at[0])
        pltpu.sync_copy(gate_hbm.at[pl.ds(r0, ROWS_PER_BLK)], gate_v.at[0])
        gathers(0, issue=True)
        stage_sg(1, 1, issue=True)

        @pl.loop(0, npairs)
        def _(t):
            process(2 * t, t, 0, 1)
            process(2 * t + 1, t, 1, 0)

        scatters(1, issue=False)
        plsc.subcore_barrier()

        @pl.loop(0, chunks_per_sub)
        def _(t):
            cidx = t * NS + s

            @pl.when(cidx < nchunks)
            def _():
                off = cidx * zrows
                pltpu.sync_copy(acc_sh.at[pl.ds(off, zrows)],
                                out_hbm.at[c].at[pl.ds(off, zrows)])

    return sc_kernel(sim, ei3, gate2)


def kernel(x, p, edge_attr, edge_index, W0, Wp, b0, W1, b1, W2, b2, We):
    E = edge_index.shape[1]
    ei3 = edge_index.reshape(2, -1, SUB)
    gate2 = _gate_tc(edge_attr.reshape(-1, SUB), We)

    sim = _sim0_tc(x, p, W0, Wp, b0)
    partial = _sc_pass(sim, ei3, gate2)
    sim = _layer_tc(partial, W1, b1)
    partial = _sc_pass(sim, ei3, gate2)
    sim = _layer_tc(partial, W2, b2)
    partial = _sc_pass(sim, ei3, gate2)
    return _sum_tc(partial)


# speedup vs baseline: 53.2772x; 1.0001x over previous
import functools

import jax
import jax.numpy as jnp
from jax import lax
from jax.experimental import pallas as pl
from jax.experimental.pallas import tpu as pltpu
from jax.experimental.pallas import tpu_sc as plsc

K = 16
NC = 2
NS = 16
NW = NC * NS
SUB = 128
ROWS_PER_BLK = 4
EB = SUB * ROWS_PER_BLK


def _gate_tc(ea2, We):
    R = ea2.shape[0]
    blk = 200
    assert R % blk == 0

    def body(ea_ref, we_ref, o_ref):
        o_ref[...] = jax.nn.sigmoid(ea_ref[...] * we_ref[0, 0])

    return pl.pallas_call(
        body,
        out_shape=jax.ShapeDtypeStruct(ea2.shape, jnp.float32),
        grid=(R // blk,),
        in_specs=[
            pl.BlockSpec((blk, 128), lambda i: (i, 0)),
            pl.BlockSpec((1, 1), lambda i: (0, 0)),
        ],
        out_specs=pl.BlockSpec((blk, 128), lambda i: (i, 0)),
    )(ea2, We)


PACK = 128 // K
_G = 25
_BR = 500


def _sim0_tc(x, p, W0, Wp, b0):
    n = x.shape[0]
    dx, dp = x.shape[1], p.shape[1]
    assert n == _G * _BR * PACK
    x4 = x.reshape(_G, _BR, PACK * dx)
    p4 = p.reshape(_G, _BR, PACK * dp)
    eye = jnp.eye(PACK, dtype=jnp.float32)
    BWx = jnp.kron(eye, W0)
    BWp = jnp.kron(eye, Wp)
    bt = jnp.tile(b0, PACK).reshape(1, PACK * K)

    def body(x_ref, p_ref, wx_ref, wp_ref, b_ref, o_ref):
        acc = jnp.dot(x_ref[0], wx_ref[...], preferred_element_type=jnp.float32)
        acc += jnp.dot(p_ref[0], wp_ref[...], preferred_element_type=jnp.float32)
        o_ref[0] = jnp.maximum(acc + b_ref[...], 0.0)

    out = pl.pallas_call(
        body,
        out_shape=jax.ShapeDtypeStruct((_G, _BR, PACK * K), jnp.float32),
        grid=(_G,),
        in_specs=[
            pl.BlockSpec((1, _BR, PACK * dx), lambda i: (i, 0, 0)),
            pl.BlockSpec((1, _BR, PACK * dp), lambda i: (i, 0, 0)),
            pl.BlockSpec(BWx.shape, lambda i: (0, 0)),
            pl.BlockSpec(BWp.shape, lambda i: (0, 0)),
            pl.BlockSpec((1, PACK * K), lambda i: (0, 0)),
        ],
        out_specs=pl.BlockSpec((1, _BR, PACK * K), lambda i: (i, 0, 0)),
    )(x4, p4, BWx, BWp, bt)
    return out.reshape(n, K)


def _layer_tc(partial, W, b):
    n = partial.shape[1]
    assert n == _G * _BR * PACK
    p4 = partial.reshape(2, _G, _BR, PACK * K)
    BW = jnp.kron(jnp.eye(PACK, dtype=jnp.float32), W)
    bt = jnp.tile(b, PACK).reshape(1, PACK * K)

    def body(p_ref, w_ref, b_ref, o_ref):
        h = p_ref[0, 0] + p_ref[1, 0]
        acc = jnp.dot(h, w_ref[...], preferred_element_type=jnp.float32)
        o_ref[0] = jnp.maximum(acc + b_ref[...], 0.0)

    out = pl.pallas_call(
        body,
        out_shape=jax.ShapeDtypeStruct((_G, _BR, PACK * K), jnp.float32),
        grid=(_G,),
        in_specs=[
            pl.BlockSpec((2, 1, _BR, PACK * K), lambda i: (0, i, 0, 0)),
            pl.BlockSpec(BW.shape, lambda i: (0, 0)),
            pl.BlockSpec((1, PACK * K), lambda i: (0, 0)),
        ],
        out_specs=pl.BlockSpec((1, _BR, PACK * K), lambda i: (i, 0, 0)),
    )(p4, BW, bt)
    return out.reshape(n, K)


def _sum_tc(partial):
    n = partial.shape[1]
    p4 = partial.reshape(2, _G, _BR, PACK * K)

    def body(p_ref, o_ref):
        o_ref[0] = p_ref[0, 0] + p_ref[1, 0]

    out = pl.pallas_call(
        body,
        out_shape=jax.ShapeDtypeStruct((_G, _BR, PACK * K), jnp.float32),
        grid=(_G,),
        in_specs=[pl.BlockSpec((2, 1, _BR, PACK * K), lambda i: (0, i, 0, 0))],
        out_specs=pl.BlockSpec((1, _BR, PACK * K), lambda i: (i, 0, 0)),
    )(p4)
    return out.reshape(n, K)


def _sc_pass(sim, ei3, gate2):
    n = sim.shape[0]
    zrows = 160
    nchunks = n // zrows
    chunks_per_sub = (nchunks + NS - 1) // NS
    R = gate2.shape[0]
    pair_rows = 2 * ROWS_PER_BLK
    rpw = -(-(R // NW) // pair_rows) * pair_rows
    last = R - (NW - 1) * rpw
    assert 0 < last <= rpw and last % pair_rows == 0
    full_pairs = rpw // pair_rows
    last_pairs = last // pair_rows
    mesh = plsc.VectorSubcoreMesh(core_axis_name="c", subcore_axis_name="s")

    @functools.partial(
        pl.kernel,
        out_type=jax.ShapeDtypeStruct((NC, n, K), jnp.float32),
        mesh=mesh,
        scratch_types=[
            pltpu.VMEM((2, ROWS_PER_BLK, SUB), jnp.int32),
            pltpu.VMEM((2, ROWS_PER_BLK, SUB), jnp.int32),
            pltpu.VMEM((2, ROWS_PER_BLK, SUB), jnp.float32),
            pltpu.VMEM((2, ROWS_PER_BLK, SUB, K), jnp.float32),
            pltpu.VMEM((zrows, K), jnp.float32),
            pltpu.VMEM_SHARED((n, K), jnp.float32),
            pltpu.SemaphoreType.DMA,
            pltpu.SemaphoreType.DMA,
            pltpu.SemaphoreType.DMA,
            pltpu.SemaphoreType.DMA,
        ],
        compiler_params=pltpu.CompilerParams(use_tc_tiling_on_sc=False),
    )
    def sc_kernel(sim_hbm, ei_hbm, gate_hbm, out_hbm,
                  src_v, dst_v, gate_v, rows_v, zbuf_v, acc_sh,
                  gsem0, gsem1, scsem, stsem):
        c = lax.axis_index("c")
        s = lax.axis_index("s")
        wid = s * NC + c
        gsems = (gsem0, gsem1)

        @pl.loop(0, zrows)
        def _(i):
            zbuf_v[i, :] = jnp.zeros((K,), jnp.float32)

        @pl.loop(0, chunks_per_sub)
        def _(t):
            cidx = t * NS + s

            @pl.when(cidx < nchunks)
            def _():
                pltpu.sync_copy(zbuf_v, acc_sh.at[pl.ds(cidx * zrows, zrows)])

        plsc.subcore_barrier()

        row_base = wid * rpw
        npairs = jnp.where(wid == NW - 1, last_pairs, full_pairs)

        def blk_r0(kblk):
            return row_base + kblk * ROWS_PER_BLK

        def stage_sg(kblk, par, issue):
            r0 = blk_r0(kblk)
            op = pltpu.async_copy if issue else pltpu.make_async_copy
            cps = [
                op(ei_hbm.at[0].at[pl.ds(r0, ROWS_PER_BLK)], src_v.at[par], stsem),
                op(gate_hbm.at[pl.ds(r0, ROWS_PER_BLK)], gate_v.at[par], stsem),
            ]
            if not issue:
                for cp in cps:
                    cp.wait()

        def stage_d(kblk, par, issue):
            r0 = blk_r0(kblk)
            op = pltpu.async_copy if issue else pltpu.make_async_copy
            cp = op(ei_hbm.at[1].at[pl.ds(r0, ROWS_PER_BLK)], dst_v.at[par], stsem)
            if not issue:
                cp.wait()

        def gathers(par, issue):
            op = pltpu.async_copy if issue else pltpu.make_async_copy
            for j in range(ROWS_PER_BLK):
                cp = op(sim_hbm.at[src_v.at[par].at[j]],
                        rows_v.at[par].at[j], gsems[par])
                if not issue:
                    cp.wait()

        def scatters(par, issue):
            for j in range(ROWS_PER_BLK):
                if issue:
                    pltpu.async_copy(rows_v.at[par].at[j],
                                     acc_sh.at[dst_v.at[par].at[j]],
                                     scsem, add=True)
                else:
                    pltpu.make_async_copy(rows_v.at[par].at[j],
                                          acc_sh.at[dst_v.at[par].at[j]],
                                          scsem).wait()

        def maybe(cond, fn):
            if cond is True:
                fn()
            else:
                pl.when(cond)(fn)

        def process(kblk, t, par, other):
            has_next = t < npairs - 1 if par else True
            has_next2 = t < npairs - 1
            not_first = True if par else t > 0

            gathers(par, issue=False)
            maybe(has_next, lambda: stage_sg(kblk + 1, other, issue=False))
            maybe(not_first, lambda: stage_d(kblk, par, issue=False))
            maybe(not_first, lambda: scatters(other, issue=False))
            maybe(has_next, lambda: stage_d(kblk + 1, other, issue=True))
            maybe(has_next, lambda: gathers(other, issue=True))

            @plsc.parallel_loop(0, ROWS_PER_BLK * (SUB // K), unroll=4)
            def _(g):
                j = g >> 3
                q = g - (j << 3)
                gv = gate_v[par, j, pl.ds(q * K, K)]
                for i in range(K):
                    b = q * K + i
                    rows_v[par, j, b, :] = rows_v[par, j, b, :] * gv[i]

            scatters(par, issue=True)
            maybe(has_next2, lambda: stage_sg(kblk + 2, par, issue=True))

        r0 = row_base
        pltpu.sync_copy(ei_hbm.at[0].at[pl.ds(r0, ROWS_PER_BLK)], src_v.at[0])
        pltpu.sync_copy(ei_hbm.at[1].at[pl.ds(r0, ROWS_PER_BLK)], dst_v.at[0])
        pltpu.sync_copy(gate_hbm.at[pl.ds(r0, ROWS_PER_BLK)], gate_v.at[0])
        gathers(0, issue=True)
        stage_sg(1, 1, issue=True)

        @pl.loop(0, npairs)
        def _(t):
            process(2 * t, t, 0, 1)
            process(2 * t + 1, t, 1, 0)

        scatters(1, issue=False)
        plsc.subcore_barrier()

        @pl.loop(0, chunks_per_sub)
        def _(t):
            cidx = t * NS + s

            @pl.when(cidx < nchunks)
            def _():
                off = cidx * zrows
                pltpu.sync_copy(acc_sh.at[pl.ds(off, zrows)],
                                out_hbm.at[c].at[pl.ds(off, zrows)])

    return sc_kernel(sim, ei3, gate2)


def kernel(x, p, edge_attr, edge_index, W0, Wp, b0, W1, b1, W2, b2, We):
    E = edge_index.shape[1]
    ei3 = edge_index.reshape(2, -1, SUB)
    gate2 = _gate_tc(edge_attr.reshape(-1, SUB), We)

    sim = _sim0_tc(x, p, W0, Wp, b0)
    partial = _sc_pass(sim, ei3, gate2)
    sim = _layer_tc(partial, W1, b1)
    partial = _sc_pass(sim, ei3, gate2)
    sim = _layer_tc(partial, W2, b2)
    partial = _sc_pass(sim, ei3, gate2)
    return _sum_tc(partial)
